# baseline (speedup 1.0000x reference)
import jax
import jax.numpy as jnp
from jax.experimental import pallas as pl


def _copy_body(x_ref, o_ref):
    o_ref[...] = x_ref[...]


def kernel(code_embeddings):
    x = code_embeddings.reshape(50000, 128)
    out = pl.pallas_call(
        _copy_body,
        out_shape=jax.ShapeDtypeStruct((50000, 128), jnp.float32),
        grid=(10,),
        in_specs=[pl.BlockSpec((5000, 128), lambda i: (i, 0))],
        out_specs=pl.BlockSpec((5000, 128), lambda i: (i, 0)),
    )(x)
    return out.reshape(100000, 64)
